# baseline (speedup 1.0000x reference)
import functools

import jax
import jax.numpy as jnp
from jax import lax
from jax.experimental import pallas as pl
from jax.experimental.pallas import tpu as pltpu
from jax.experimental.pallas import tpu_sc as plsc

_NUM_EMB = 1024
_DIM = 32
_COMMITMENT = 0.25
_N = 32 * 2048
_BLOCK = 2048
_GRID = _N // _BLOCK
_ASUB = 8


_SC_WORKERS = 32
_SC_ROWS = _N // _SC_WORKERS
_SC_CHUNK = 64


def _enc_zero_body(z_hbm, out_hbm, zbuf, sem):
    wid = lax.axis_index("s") * 2 + lax.axis_index("c")
    base = wid * _SC_ROWS
    pltpu.sync_copy(z_hbm, zbuf)
    copies = [
        pltpu.async_copy(zbuf, out_hbm.at[pl.ds(base + j * _SC_CHUNK, _SC_CHUNK)], sem)
        for j in range(_SC_ROWS // _SC_CHUNK)
    ]
    for cp in copies:
        cp.wait()


def _enc_zero(zrows):
    mesh = plsc.VectorSubcoreMesh(core_axis_name="c", subcore_axis_name="s")
    return functools.partial(
        pl.kernel, mesh=mesh,
        out_type=jax.ShapeDtypeStruct((_N, _NUM_EMB), jnp.float32),
        scratch_types=[
            pltpu.VMEM((_SC_CHUNK, _NUM_EMB), jnp.float32),
            pltpu.SemaphoreType.DMA,
        ],
    )(_enc_zero_body)(zrows)


def _vq_body(x_ref, e_ref, dist_ref,
             quant_ref, idx_ref, loss_ref, perp_ref, acc_ref, cnt_ref):
    step = pl.program_id(0)
    al = step % _ASUB

    xt = x_ref[:, al, :]
    e = e_ref[...]
    em2 = -2.0 * e
    x2 = jnp.sum(xt * xt, axis=0)[:, None]
    e2 = jnp.sum(e * e, axis=1)[None, :]
    xem2 = jax.lax.dot_general(xt, em2, (((0,), (1,)), ((), ())),
                               preferred_element_type=jnp.float32)
    dist = (x2 + e2) + xem2
    dist_ref[...] = dist

    mn = jnp.min(dist, axis=1, keepdims=True)
    lane = jax.lax.broadcasted_iota(jnp.int32, (_BLOCK, _NUM_EMB), 1)
    idx = jnp.min(jnp.where(dist == mn, lane, _NUM_EMB), axis=1)
    onehot = (lane == idx[:, None]).astype(jnp.float32)
    idx_ref[...] = idx[:, None]

    quant_t = jax.lax.dot_general(e, onehot, (((0,), (1,)), ((), ())),
                                  preferred_element_type=jnp.float32)
    quant_ref[:, al, :] = quant_t

    ones = jnp.ones((1, _BLOCK), jnp.float32)
    cnt = jax.lax.dot_general(ones, onehot, (((1,), (0,)), ((), ())),
                              preferred_element_type=jnp.float32)
    sse = jnp.sum(mn).reshape(1, 1)

    @pl.when(step == 0)
    def _init():
        acc_ref[...] = sse
        cnt_ref[...] = cnt

    @pl.when(step != 0)
    def _accum():
        acc_ref[...] += sse
        cnt_ref[...] += cnt

    @pl.when(step == _GRID - 1)
    def _finalize():
        mse = acc_ref[...] * (1.0 / (_N * _DIM))
        loss_ref[...] = (1.0 + _COMMITMENT) * mse
        probs = cnt_ref[...] * (1.0 / _N)
        ent = jnp.sum(probs * jnp.log(probs + 1e-10))
        perp_ref[...] = jnp.exp(-ent).reshape(1, 1)


def kernel(inputs, embedding_weight):
    enc0 = _enc_zero(jnp.zeros((_SC_CHUNK, _NUM_EMB), jnp.float32))

    dist, quant_t, idx, loss, perp = pl.pallas_call(
        _vq_body,
        grid=(_GRID,),
        in_specs=[
            pl.BlockSpec((_DIM, _ASUB, _BLOCK), lambda i: (0, i // _ASUB, 0)),
            pl.BlockSpec((_NUM_EMB, _DIM), lambda i: (0, 0)),
        ],
        out_specs=[
            pl.BlockSpec((_BLOCK, _NUM_EMB), lambda i: (i, 0)),
            pl.BlockSpec((_DIM, _ASUB, _BLOCK), lambda i: (0, i // _ASUB, 0)),
            pl.BlockSpec((_BLOCK, 1), lambda i: (i, 0)),
            pl.BlockSpec((1, 1), lambda i: (0, 0)),
            pl.BlockSpec((1, 1), lambda i: (0, 0)),
        ],
        out_shape=[
            jax.ShapeDtypeStruct((_N, _NUM_EMB), jnp.float32),
            jax.ShapeDtypeStruct((_DIM, 32, _BLOCK), jnp.float32),
            jax.ShapeDtypeStruct((_N, 1), jnp.int32),
            jax.ShapeDtypeStruct((1, 1), jnp.float32),
            jax.ShapeDtypeStruct((1, 1), jnp.float32),
        ],
        scratch_shapes=[
            pltpu.VMEM((1, 1), jnp.float32),
            pltpu.VMEM((1, _NUM_EMB), jnp.float32),
        ],
    )(inputs, embedding_weight)

    rows = jnp.arange(_N, dtype=jnp.int32)
    enc = enc0.at[rows, idx[:, 0]].set(1.0, unique_indices=True)

    return (loss[0, 0],
            quant_t,
            perp[0, 0],
            enc.reshape(32, 2048, _NUM_EMB),
            dist.reshape(32, 2048, _NUM_EMB),
            idx)


# speedup vs baseline: 5.2292x; 5.2292x over previous
import jax
import jax.numpy as jnp
from jax.experimental import pallas as pl
from jax.experimental.pallas import tpu as pltpu

_NUM_EMB = 1024
_DIM = 32
_COMMITMENT = 0.25
_N = 32 * 2048
_BLOCK = 2048
_GRID = _N // _BLOCK
_ASUB = 8


def _vq_body(x_ref, e_ref, dist_ref, enc_ref,
             quant_ref, idx_ref, loss_ref, perp_ref, acc_ref, cnt_ref):
    step = pl.program_id(0)
    al = step % _ASUB

    xt = x_ref[:, al, :]
    e = e_ref[...]
    em2 = -2.0 * e
    x2 = jnp.sum(xt * xt, axis=0)[:, None]
    e2 = jnp.sum(e * e, axis=1)[None, :]
    xem2 = jax.lax.dot_general(xt, em2, (((0,), (1,)), ((), ())),
                               preferred_element_type=jnp.float32)
    dist = (x2 + e2) + xem2
    dist_ref[...] = dist

    mn = jnp.min(dist, axis=1, keepdims=True)
    lane = jax.lax.broadcasted_iota(jnp.int32, (_BLOCK, _NUM_EMB), 1)
    idx = jnp.min(jnp.where(dist == mn, lane, _NUM_EMB), axis=1)
    onehot = (lane == idx[:, None]).astype(jnp.float32)
    enc_ref[...] = onehot
    idx_ref[...] = idx[:, None]

    quant_t = jax.lax.dot_general(e, onehot, (((0,), (1,)), ((), ())),
                                  preferred_element_type=jnp.float32)
    quant_ref[:, al, :] = quant_t

    ones = jnp.ones((1, _BLOCK), jnp.float32)
    cnt = jax.lax.dot_general(ones, onehot, (((1,), (0,)), ((), ())),
                              preferred_element_type=jnp.float32)
    sse = jnp.sum(mn).reshape(1, 1)

    @pl.when(step == 0)
    def _init():
        acc_ref[...] = sse
        cnt_ref[...] = cnt

    @pl.when(step != 0)
    def _accum():
        acc_ref[...] += sse
        cnt_ref[...] += cnt

    @pl.when(step == _GRID - 1)
    def _finalize():
        mse = acc_ref[...] * (1.0 / (_N * _DIM))
        loss_ref[...] = (1.0 + _COMMITMENT) * mse
        probs = cnt_ref[...] * (1.0 / _N)
        ent = jnp.sum(probs * jnp.log(probs + 1e-10))
        perp_ref[...] = jnp.exp(-ent).reshape(1, 1)


def kernel(inputs, embedding_weight):
    dist, enc, quant_t, idx, loss, perp = pl.pallas_call(
        _vq_body,
        grid=(_GRID,),
        in_specs=[
            pl.BlockSpec((_DIM, _ASUB, _BLOCK), lambda i: (0, i // _ASUB, 0)),
            pl.BlockSpec((_NUM_EMB, _DIM), lambda i: (0, 0)),
        ],
        out_specs=[
            pl.BlockSpec((_BLOCK, _NUM_EMB), lambda i: (i, 0)),
            pl.BlockSpec((_BLOCK, _NUM_EMB), lambda i: (i, 0)),
            pl.BlockSpec((_DIM, _ASUB, _BLOCK), lambda i: (0, i // _ASUB, 0)),
            pl.BlockSpec((_BLOCK, 1), lambda i: (i, 0)),
            pl.BlockSpec((1, 1), lambda i: (0, 0)),
            pl.BlockSpec((1, 1), lambda i: (0, 0)),
        ],
        out_shape=[
            jax.ShapeDtypeStruct((_N, _NUM_EMB), jnp.float32),
            jax.ShapeDtypeStruct((_N, _NUM_EMB), jnp.float32),
            jax.ShapeDtypeStruct((_DIM, 32, _BLOCK), jnp.float32),
            jax.ShapeDtypeStruct((_N, 1), jnp.int32),
            jax.ShapeDtypeStruct((1, 1), jnp.float32),
            jax.ShapeDtypeStruct((1, 1), jnp.float32),
        ],
        scratch_shapes=[
            pltpu.VMEM((1, 1), jnp.float32),
            pltpu.VMEM((1, _NUM_EMB), jnp.float32),
        ],
    )(inputs, embedding_weight)

    return (loss[0, 0],
            quant_t,
            perp[0, 0],
            enc.reshape(32, 2048, _NUM_EMB),
            dist.reshape(32, 2048, _NUM_EMB),
            idx)
